# baseline (speedup 1.0000x reference)
import jax
import jax.numpy as jnp
from jax.experimental import pallas as pl
from jax.experimental.pallas import tpu as pltpu

_J = 128
_D = 4096
_BIAS = 0.01
_TILE = 512
_NG = _D // _TILE

_CONTRACT_RHS1 = (((1,), (1,)), ((), ()))


def _cell_kernel(s_ref, hb_ref, keys_ref, u_ref, v_ref, w_ref, a_ref,
                 o_ref, g_ref, ss_ref):
    i = pl.program_id(0)

    @pl.when(i == 0)
    def _():
        z = jnp.sum((hb_ref[...] + keys_ref[...]) * s_ref[...],
                    axis=1, keepdims=True)
        g_ref[...] = jax.nn.sigmoid(z)
        ss_ref[...] = jnp.zeros_like(ss_ref)

    pre = jax.lax.dot_general(hb_ref[...], u_ref[...], _CONTRACT_RHS1,
                              preferred_element_type=jnp.float32)
    pre = pre + jax.lax.dot_general(keys_ref[...], v_ref[...], _CONTRACT_RHS1,
                                    preferred_element_type=jnp.float32)
    ws = jax.lax.dot_general(s_ref[...], w_ref[...], _CONTRACT_RHS1,
                             preferred_element_type=jnp.float32)
    pre = pre + ws + _BIAS
    cand = jnp.where(pre >= 0.0, pre, a_ref[...] * pre)
    sl = pl.ds(i * _TILE, _TILE)
    tile = hb_ref[:, sl] + g_ref[...] * cand
    o_ref[:, sl] = tile
    ss_ref[...] += jnp.sum(tile * tile, axis=1, keepdims=True)

    @pl.when(i == _NG - 1)
    def _():
        r = 1.0 / jnp.sqrt(ss_ref[...])
        o_ref[...] = o_ref[...] * r


def kernel(s, h, keys, U, V, W, prelu_a):
    hb = h.reshape(_J, _D)
    s2 = s.reshape(1, _D)
    a2 = prelu_a.reshape(1, 1)

    full = lambda shape: pl.BlockSpec(shape, lambda i: (0,) * len(shape))
    tile_rows = pl.BlockSpec((_TILE, _D), lambda i: (i, 0))

    out = pl.pallas_call(
        _cell_kernel,
        out_shape=jax.ShapeDtypeStruct((_J, _D), jnp.float32),
        grid=(_NG,),
        in_specs=[
            full((1, _D)),
            full((_J, _D)),
            full((_J, _D)),
            tile_rows,
            tile_rows,
            tile_rows,
            full((1, 1)),
        ],
        out_specs=full((_J, _D)),
        scratch_shapes=[pltpu.VMEM((_J, 1), jnp.float32),
                        pltpu.VMEM((_J, 1), jnp.float32)],
        compiler_params=pltpu.CompilerParams(
            dimension_semantics=("arbitrary",),
            vmem_limit_bytes=57 * 1024 * 1024,
        ),
        name="memory_cell_fused",
    )(s2, hb, keys, U, V, W, a2)

    return out.reshape(-1)


# speedup vs baseline: 1.0491x; 1.0491x over previous
import jax
import jax.numpy as jnp
from jax.experimental import pallas as pl
from jax.experimental.pallas import tpu as pltpu

_J = 128
_D = 4096
_BIAS = 0.01
_TILE = 256
_NG = _D // _TILE

_CONTRACT_RHS1 = (((1,), (1,)), ((), ()))


def _cell_kernel(s_ref, hb_ref, keys_ref, u_ref, v_ref, w_ref, a_ref,
                 o_ref, g_ref, ss_ref):
    i = pl.program_id(0)

    @pl.when(i == 0)
    def _():
        z = jnp.sum((hb_ref[...] + keys_ref[...]) * s_ref[...],
                    axis=1, keepdims=True)
        g_ref[...] = jax.nn.sigmoid(z)
        ss_ref[...] = jnp.zeros_like(ss_ref)

    pre = jax.lax.dot_general(hb_ref[...], u_ref[...], _CONTRACT_RHS1,
                              preferred_element_type=jnp.float32)
    pre = pre + jax.lax.dot_general(keys_ref[...], v_ref[...], _CONTRACT_RHS1,
                                    preferred_element_type=jnp.float32)
    ws = jax.lax.dot_general(s_ref[...], w_ref[...], _CONTRACT_RHS1,
                             preferred_element_type=jnp.float32)
    pre = pre + ws + _BIAS
    cand = jnp.where(pre >= 0.0, pre, a_ref[...] * pre)
    sl = pl.ds(i * _TILE, _TILE)
    tile = hb_ref[:, sl] + g_ref[...] * cand
    o_ref[:, sl] = tile
    ss_ref[...] += jnp.sum(tile * tile, axis=1, keepdims=True)

    @pl.when(i == _NG - 1)
    def _():
        r = 1.0 / jnp.sqrt(ss_ref[...])
        o_ref[...] = o_ref[...] * r


def kernel(s, h, keys, U, V, W, prelu_a):
    hb = h.reshape(_J, _D)
    s2 = s.reshape(1, _D)
    a2 = prelu_a.reshape(1, 1)

    full = lambda shape: pl.BlockSpec(shape, lambda i: (0,) * len(shape))
    tile_rows = pl.BlockSpec((_TILE, _D), lambda i: (i, 0))

    out = pl.pallas_call(
        _cell_kernel,
        out_shape=jax.ShapeDtypeStruct((_J, _D), jnp.float32),
        grid=(_NG,),
        in_specs=[
            full((1, _D)),
            full((_J, _D)),
            full((_J, _D)),
            tile_rows,
            tile_rows,
            tile_rows,
            full((1, 1)),
        ],
        out_specs=full((_J, _D)),
        scratch_shapes=[pltpu.VMEM((_J, 1), jnp.float32),
                        pltpu.VMEM((_J, 1), jnp.float32)],
        compiler_params=pltpu.CompilerParams(
            dimension_semantics=("arbitrary",),
            vmem_limit_bytes=57 * 1024 * 1024,
        ),
        name="memory_cell_fused",
    )(s2, hb, keys, U, V, W, a2)

    return out.reshape(-1)
